# baseline (speedup 1.0000x reference)
import functools

import jax
import jax.numpy as jnp
from jax import lax
from jax.experimental import pallas as pl
from jax.experimental.pallas import tpu as pltpu
from jax.experimental.pallas import tpu_sc as plsc

EPS = 1e-5

NC = 2
NS = 16
NW = NC * NS

CH = 250
K = 40
GRP = 25
CPG = CH // GRP


NPAD = 10240


def _make_agg(width, with_cnt, NBUF):
    LAG = NBUF - 1
    rows_per_sub = NPAD // NS
    mesh = plsc.VectorSubcoreMesh(core_axis_name="c", subcore_axis_name="s")

    out_type = [jax.ShapeDtypeStruct((NC, NPAD, width), jnp.float32)]
    scratch = [pltpu.VMEM((CPG, K), jnp.int32) for _ in range(4)]
    scratch += [pltpu.VMEM((K, width), jnp.float32) for _ in range(NBUF)]
    scratch += [pltpu.VMEM_SHARED((NPAD, width), jnp.float32)]
    scratch += [pltpu.SemaphoreType.DMA for _ in range(2 * NBUF + 2)]
    if with_cnt:
        out_type.append(jax.ShapeDtypeStruct((NC, NS, NPAD), jnp.float32))
        scratch.append(pltpu.VMEM((NPAD,), jnp.float32))

    @functools.partial(
        pl.kernel,
        out_type=out_type,
        mesh=mesh,
        scratch_types=scratch,
        compiler_params=pltpu.CompilerParams(needs_layout_passes=False),
    )
    def agg(tbl, src3, dst3, zdrain, out, *rest):
        if with_cnt:
            cnt_out = rest[0]
            rest = rest[1:]
        srcv = rest[0:2]
        dstv = rest[2:4]
        rows = rest[4:4 + NBUF]
        acc = rest[4 + NBUF]
        sg = rest[5 + NBUF:5 + 2 * NBUF]
        ss = rest[5 + 2 * NBUF:5 + 3 * NBUF]
        si = rest[5 + 3 * NBUF:7 + 3 * NBUF]
        if with_cnt:
            cntloc = rest[7 + 3 * NBUF]
        c = lax.axis_index("c")
        s = lax.axis_index("s")
        wid = c * NS + s
        zv = jnp.zeros((16,), jnp.float32)

        def zrow(r, carry):
            for cc in range(width // 16):
                rows[0][r, pl.ds(cc * 16, 16)] = zv
            return carry

        lax.fori_loop(0, K, zrow, 0)
        for kk in range(rows_per_sub // K):
            pltpu.sync_copy(
                rows[0], acc.at[pl.ds(s * rows_per_sub + kk * K, K)]
            )
        if with_cnt:

            def zbody(i, carry):
                cntloc[pl.ds(i * 16, 16)] = zv
                return carry

            lax.fori_loop(0, NPAD // 16, zbody, 0)
        plsc.subcore_barrier()

        ones16 = jnp.ones((16,), jnp.float32)

        tmask = lax.iota(jnp.int32, 16) >= 8

        def count16(dv, i):
            if with_cnt:
                for j in range(2):
                    d16 = dv[i, pl.ds(j * 16, 16)]
                    plsc.addupdate_scatter(cntloc, [d16], ones16)
                dt = dv[i, pl.ds(K - 16, 16)]
                plsc.addupdate_scatter(cntloc, [dt], ones16, mask=tmask)

        def drain_scatter(b):
            pltpu.make_async_copy(zdrain, rows[b], ss[b]).wait()

        def scatter(cc, gdesc):
            gdesc[cc].wait()
            gset = (cc // CPG) % 2
            pltpu.async_copy(
                rows[cc % NBUF], acc.at[dstv[gset].at[cc % CPG]],
                ss[cc % NBUF], add=True,
            )

        pltpu.sync_copy(src3.at[wid, 0], srcv[0])
        pltpu.sync_copy(dst3.at[wid, 0], dstv[0])
        gdesc = [None] * CH
        pf = None
        for g in range(GRP):
            cur = g % 2
            if pf is not None:
                pf[0].wait()
                pf[1].wait()
            pf = None
            for k in range(CPG):
                cc = g * CPG + k
                b = cc % NBUF
                if cc >= NBUF:
                    drain_scatter(b)
                gdesc[cc] = pltpu.async_copy(
                    tbl.at[srcv[cur].at[k]], rows[b], sg[b]
                )
                count16(dstv[cur], k)
                if cc >= LAG:
                    scatter(cc - LAG, gdesc)
            if g + 1 < GRP:
                nxt = 1 - cur
                pf = (
                    pltpu.async_copy(src3.at[wid, g + 1], srcv[nxt], si[0]),
                    pltpu.async_copy(dst3.at[wid, g + 1], dstv[nxt], si[1]),
                )
        for cc in range(CH - LAG, CH):
            scatter(cc, gdesc)
        for b in range(NBUF):
            drain_scatter(b)
        plsc.subcore_barrier()
        sl = pl.ds(s * rows_per_sub, rows_per_sub)
        pltpu.sync_copy(acc.at[sl], out.at[c, sl])
        if with_cnt:
            pltpu.sync_copy(cntloc, cnt_out.at[c, s])

    return agg


def _mid_body(p_ref, cntp_ref, w_ref, b_ref, g_ref, be_ref, y_ref, cnt_ref):
    n = y_ref.shape[0]
    sx = p_ref[0, :n, :] + p_ref[1, :n, :]
    dn = (((0,), (0,)), ((), ()))
    nw = cntp_ref.shape[0]
    cnt_full = lax.dot_general(
        cntp_ref[...],
        jnp.ones((nw, 1), jnp.float32),
        dn,
        preferred_element_type=jnp.float32,
    )
    cnt = cnt_full[:n, :]
    sy = jnp.dot(sx, w_ref[...], preferred_element_type=jnp.float32)
    h = (sy + cnt * b_ref[...]) / jnp.maximum(cnt, 1.0)
    m = jnp.mean(h, axis=0, keepdims=True)
    v = jnp.mean((h - m) ** 2, axis=0, keepdims=True)
    hn = (h - m) * lax.rsqrt(v + EPS) * g_ref[...] + be_ref[...]
    y_ref[...] = jnp.where(hn > 0, hn, jnp.exp(jnp.minimum(hn, 0.0)) - 1.0)
    cnt_ref[...] = cnt


def _final_body(p_ref, cnt_ref, w_ref, b_ref, g_ref, be_ref, batch_ref, wo_ref, bo_ref, o_ref):
    nn = batch_ref.shape[0]
    sx = p_ref[0, :nn, :] + p_ref[1, :nn, :]
    cnt = cnt_ref[...]
    sy = jnp.dot(sx, w_ref[...], preferred_element_type=jnp.float32)
    h = (sy + cnt * b_ref[...]) / jnp.maximum(cnt, 1.0)
    m = jnp.mean(h, axis=0, keepdims=True)
    v = jnp.mean((h - m) ** 2, axis=0, keepdims=True)
    hn = (h - m) * lax.rsqrt(v + EPS) * g_ref[...] + be_ref[...]
    h2 = jnp.where(hn > 0, hn, jnp.exp(jnp.minimum(hn, 0.0)) - 1.0)
    g = o_ref.shape[0]
    oh = (batch_ref[...] == lax.broadcasted_iota(jnp.int32, (nn, g), 1))
    oh = oh.astype(jnp.float32)
    dn = (((0,), (0,)), ((), ()))
    ps = lax.dot_general(oh, h2, dn, preferred_element_type=jnp.float32)
    pc = lax.dot_general(
        oh, jnp.ones((nn, 1), jnp.float32), dn, preferred_element_type=jnp.float32
    )
    pooled = ps / jnp.maximum(pc, 1.0)
    out = jnp.dot(pooled, wo_ref[...], preferred_element_type=jnp.float32)
    o_ref[...] = out + bo_ref[...]


def kernel(data, edge_index, batch, W1, b1, g1, be1, W2, b2, g2, be2, Wout, bout):
    n, d = data.shape
    h_dim = W1.shape[1]
    out_dim = Wout.shape[1]
    g_graphs = 64

    src3 = edge_index[0].reshape(NW, GRP, CPG, K)
    dst3 = edge_index[1].reshape(NW, GRP, CPG, K)
    zdrain = jnp.zeros((K, h_dim), jnp.float32)

    p1, cntp = _make_agg(h_dim, True, 6)(data, src3, dst3, zdrain)

    x2, cnt = pl.pallas_call(
        _mid_body,
        out_shape=[
            jax.ShapeDtypeStruct((n, h_dim), jnp.float32),
            jax.ShapeDtypeStruct((n, 1), jnp.float32),
        ],
    )(
        p1,
        cntp.reshape(NW, NPAD),
        W1,
        b1.reshape(1, h_dim),
        g1.reshape(1, h_dim),
        be1.reshape(1, h_dim),
    )

    (p2,) = _make_agg(h_dim, False, 8)(x2, src3, dst3, zdrain)

    out = pl.pallas_call(
        _final_body,
        out_shape=jax.ShapeDtypeStruct((g_graphs, out_dim), jnp.float32),
    )(
        p2,
        cnt,
        W2,
        b2.reshape(1, h_dim),
        g2.reshape(1, h_dim),
        be2.reshape(1, h_dim),
        batch.reshape(n, 1),
        Wout,
        bout.reshape(1, out_dim),
    )
    return out


# speedup vs baseline: 1.0308x; 1.0308x over previous
import functools

import jax
import jax.numpy as jnp
from jax import lax
from jax.experimental import pallas as pl
from jax.experimental.pallas import tpu as pltpu
from jax.experimental.pallas import tpu_sc as plsc

EPS = 1e-5

NC = 2
NS = 16
NW = NC * NS

K1 = 80
K2 = 40
GRP = 25


NPAD = 10240


def _make_agg(width, with_cnt, NBUF, K):
    LAG = NBUF - 1
    CH = 10000 // K
    CPG = CH // GRP
    rows_per_sub = NPAD // NS
    mesh = plsc.VectorSubcoreMesh(core_axis_name="c", subcore_axis_name="s")

    out_type = [jax.ShapeDtypeStruct((NC, NPAD, width), jnp.float32)]
    scratch = [pltpu.VMEM((CPG, K), jnp.int32) for _ in range(4)]
    scratch += [pltpu.VMEM((K, width), jnp.float32) for _ in range(NBUF)]
    scratch += [pltpu.VMEM_SHARED((NPAD, width), jnp.float32)]
    scratch += [pltpu.SemaphoreType.DMA for _ in range(2 * NBUF + 2)]
    if with_cnt:
        out_type.append(jax.ShapeDtypeStruct((NC, NS, NPAD), jnp.float32))
        scratch.append(pltpu.VMEM((NPAD,), jnp.float32))

    @functools.partial(
        pl.kernel,
        out_type=out_type,
        mesh=mesh,
        scratch_types=scratch,
        compiler_params=pltpu.CompilerParams(needs_layout_passes=False),
    )
    def agg(tbl, src3, dst3, zdrain, out, *rest):
        if with_cnt:
            cnt_out = rest[0]
            rest = rest[1:]
        srcv = rest[0:2]
        dstv = rest[2:4]
        rows = rest[4:4 + NBUF]
        acc = rest[4 + NBUF]
        sg = rest[5 + NBUF:5 + 2 * NBUF]
        ss = rest[5 + 2 * NBUF:5 + 3 * NBUF]
        si = rest[5 + 3 * NBUF:7 + 3 * NBUF]
        if with_cnt:
            cntloc = rest[7 + 3 * NBUF]
        c = lax.axis_index("c")
        s = lax.axis_index("s")
        wid = c * NS + s
        zv = jnp.zeros((16,), jnp.float32)

        def zrow(r, carry):
            for cc in range(width // 16):
                rows[0][r, pl.ds(cc * 16, 16)] = zv
            return carry

        lax.fori_loop(0, K, zrow, 0)
        for kk in range(rows_per_sub // K):
            pltpu.sync_copy(
                rows[0], acc.at[pl.ds(s * rows_per_sub + kk * K, K)]
            )
        if with_cnt:

            def zbody(i, carry):
                cntloc[pl.ds(i * 16, 16)] = zv
                return carry

            lax.fori_loop(0, NPAD // 16, zbody, 0)
        plsc.subcore_barrier()

        ones16 = jnp.ones((16,), jnp.float32)

        tmask = lax.iota(jnp.int32, 16) >= (16 - K % 16)

        def count16(dv, i):
            if with_cnt:
                for j in range(K // 16):
                    d16 = dv[i, pl.ds(j * 16, 16)]
                    plsc.addupdate_scatter(cntloc, [d16], ones16)
                if K % 16:
                    dt = dv[i, pl.ds(K - 16, 16)]
                    plsc.addupdate_scatter(cntloc, [dt], ones16, mask=tmask)

        def drain_scatter(b):
            pltpu.make_async_copy(zdrain.at[pl.ds(0, K)], rows[b], ss[b]).wait()

        def scatter(cc, gdesc):
            gdesc[cc].wait()
            gset = (cc // CPG) % 2
            pltpu.async_copy(
                rows[cc % NBUF], acc.at[dstv[gset].at[cc % CPG]],
                ss[cc % NBUF], add=True,
            )

        pltpu.sync_copy(src3.at[wid, 0], srcv[0])
        pltpu.sync_copy(dst3.at[wid, 0], dstv[0])
        gdesc = [None] * CH
        pf = None
        for g in range(GRP):
            cur = g % 2
            if pf is not None:
                pf[0].wait()
                pf[1].wait()
            pf = None
            for k in range(CPG):
                cc = g * CPG + k
                b = cc % NBUF
                if cc >= NBUF:
                    drain_scatter(b)
                gdesc[cc] = pltpu.async_copy(
                    tbl.at[srcv[cur].at[k]], rows[b], sg[b]
                )
                count16(dstv[cur], k)
                if cc >= LAG:
                    scatter(cc - LAG, gdesc)
            if g + 1 < GRP:
                nxt = 1 - cur
                pf = (
                    pltpu.async_copy(src3.at[wid, g + 1], srcv[nxt], si[0]),
                    pltpu.async_copy(dst3.at[wid, g + 1], dstv[nxt], si[1]),
                )
        for cc in range(CH - LAG, CH):
            scatter(cc, gdesc)
        for b in range(NBUF):
            drain_scatter(b)
        plsc.subcore_barrier()
        sl = pl.ds(s * rows_per_sub, rows_per_sub)
        pltpu.sync_copy(acc.at[sl], out.at[c, sl])
        if with_cnt:
            pltpu.sync_copy(cntloc, cnt_out.at[c, s])

    return agg


def _mid_body(p_ref, cntp_ref, w_ref, b_ref, g_ref, be_ref, y_ref, cnt_ref):
    n = y_ref.shape[0]
    sx = p_ref[0, :n, :] + p_ref[1, :n, :]
    dn = (((0,), (0,)), ((), ()))
    nw = cntp_ref.shape[0]
    cnt_full = lax.dot_general(
        cntp_ref[...],
        jnp.ones((nw, 1), jnp.float32),
        dn,
        preferred_element_type=jnp.float32,
    )
    cnt = cnt_full[:n, :]
    sy = jnp.dot(sx, w_ref[...], preferred_element_type=jnp.float32)
    h = (sy + cnt * b_ref[...]) / jnp.maximum(cnt, 1.0)
    m = jnp.mean(h, axis=0, keepdims=True)
    v = jnp.mean((h - m) ** 2, axis=0, keepdims=True)
    hn = (h - m) * lax.rsqrt(v + EPS) * g_ref[...] + be_ref[...]
    y_ref[...] = jnp.where(hn > 0, hn, jnp.exp(jnp.minimum(hn, 0.0)) - 1.0)
    cnt_ref[...] = cnt


def _final_body(p_ref, cnt_ref, w_ref, b_ref, g_ref, be_ref, batch_ref, wo_ref, bo_ref, o_ref):
    nn = batch_ref.shape[0]
    sx = p_ref[0, :nn, :] + p_ref[1, :nn, :]
    cnt = cnt_ref[...]
    sy = jnp.dot(sx, w_ref[...], preferred_element_type=jnp.float32)
    h = (sy + cnt * b_ref[...]) / jnp.maximum(cnt, 1.0)
    m = jnp.mean(h, axis=0, keepdims=True)
    v = jnp.mean((h - m) ** 2, axis=0, keepdims=True)
    hn = (h - m) * lax.rsqrt(v + EPS) * g_ref[...] + be_ref[...]
    h2 = jnp.where(hn > 0, hn, jnp.exp(jnp.minimum(hn, 0.0)) - 1.0)
    g = o_ref.shape[0]
    oh = (batch_ref[...] == lax.broadcasted_iota(jnp.int32, (nn, g), 1))
    oh = oh.astype(jnp.float32)
    dn = (((0,), (0,)), ((), ()))
    ps = lax.dot_general(oh, h2, dn, preferred_element_type=jnp.float32)
    pc = lax.dot_general(
        oh, jnp.ones((nn, 1), jnp.float32), dn, preferred_element_type=jnp.float32
    )
    pooled = ps / jnp.maximum(pc, 1.0)
    out = jnp.dot(pooled, wo_ref[...], preferred_element_type=jnp.float32)
    o_ref[...] = out + bo_ref[...]


def kernel(data, edge_index, batch, W1, b1, g1, be1, W2, b2, g2, be2, Wout, bout):
    n, d = data.shape
    h_dim = W1.shape[1]
    out_dim = Wout.shape[1]
    g_graphs = 64

    src1 = edge_index[0].reshape(NW, GRP, 10000 // K1 // GRP, K1)
    dst1 = edge_index[1].reshape(NW, GRP, 10000 // K1 // GRP, K1)
    src2 = edge_index[0].reshape(NW, GRP, 10000 // K2 // GRP, K2)
    dst2 = edge_index[1].reshape(NW, GRP, 10000 // K2 // GRP, K2)
    zdrain = jnp.zeros((max(K1, K2), h_dim), jnp.float32)

    p1, cntp = _make_agg(h_dim, True, 3, K1)(data, src1, dst1, zdrain)

    x2, cnt = pl.pallas_call(
        _mid_body,
        out_shape=[
            jax.ShapeDtypeStruct((n, h_dim), jnp.float32),
            jax.ShapeDtypeStruct((n, 1), jnp.float32),
        ],
    )(
        p1,
        cntp.reshape(NW, NPAD),
        W1,
        b1.reshape(1, h_dim),
        g1.reshape(1, h_dim),
        be1.reshape(1, h_dim),
    )

    (p2,) = _make_agg(h_dim, False, 8, K2)(x2, src2, dst2, zdrain)

    out = pl.pallas_call(
        _final_body,
        out_shape=jax.ShapeDtypeStruct((g_graphs, out_dim), jnp.float32),
    )(
        p2,
        cnt,
        W2,
        b2.reshape(1, h_dim),
        g2.reshape(1, h_dim),
        be2.reshape(1, h_dim),
        batch.reshape(n, 1),
        Wout,
        bout.reshape(1, out_dim),
    )
    return out


# speedup vs baseline: 1.1024x; 1.0695x over previous
import functools

import jax
import jax.numpy as jnp
from jax import lax
from jax.experimental import pallas as pl
from jax.experimental.pallas import tpu as pltpu
from jax.experimental.pallas import tpu_sc as plsc

EPS = 1e-5

NC = 2
NS = 16
NW = NC * NS

CH = 125
K = 80
GRP = 25
CPG = CH // GRP


NPAD = 10240


def _make_agg(width, with_cnt, NBUF):
    LAG = NBUF - 1
    rows_per_sub = NPAD // NS
    mesh = plsc.VectorSubcoreMesh(core_axis_name="c", subcore_axis_name="s")

    out_type = [jax.ShapeDtypeStruct((NC, NPAD, width), jnp.float32)]
    scratch = [pltpu.VMEM((CPG, K), jnp.int32) for _ in range(4)]
    scratch += [pltpu.VMEM((K, width), jnp.float32) for _ in range(NBUF)]
    scratch += [pltpu.VMEM_SHARED((NPAD, width), jnp.float32)]
    scratch += [pltpu.SemaphoreType.DMA for _ in range(2 * NBUF + 2)]
    if with_cnt:
        out_type.append(jax.ShapeDtypeStruct((NC, NS, NPAD), jnp.float32))
        scratch.append(pltpu.VMEM((NPAD,), jnp.float32))

    @functools.partial(
        pl.kernel,
        out_type=out_type,
        mesh=mesh,
        scratch_types=scratch,
        compiler_params=pltpu.CompilerParams(needs_layout_passes=False),
    )
    def agg(tbl, e5, zdrain, out, *rest):
        if with_cnt:
            cnt_out = rest[0]
            rest = rest[1:]
        srcv = rest[0:2]
        dstv = rest[2:4]
        rows = rest[4:4 + NBUF]
        acc = rest[4 + NBUF]
        sg = rest[5 + NBUF:5 + 2 * NBUF]
        ss = rest[5 + 2 * NBUF:5 + 3 * NBUF]
        si = rest[5 + 3 * NBUF:7 + 3 * NBUF]
        if with_cnt:
            cntloc = rest[7 + 3 * NBUF]
        c = lax.axis_index("c")
        s = lax.axis_index("s")
        wid = c * NS + s
        zv = jnp.zeros((16,), jnp.float32)

        def zrow(r, carry):
            for cc in range(width // 16):
                rows[0][r, pl.ds(cc * 16, 16)] = zv
            return carry

        lax.fori_loop(0, K, zrow, 0)
        for kk in range(rows_per_sub // K):
            pltpu.sync_copy(
                rows[0], acc.at[pl.ds(s * rows_per_sub + kk * K, K)]
            )
        if with_cnt:

            def zbody(i, carry):
                cntloc[pl.ds(i * 16, 16)] = zv
                return carry

            lax.fori_loop(0, NPAD // 16, zbody, 0)
        plsc.subcore_barrier()

        ones16 = jnp.ones((16,), jnp.float32)

        def count16(dv, i):
            if with_cnt:
                for j in range(K // 16):
                    d16 = dv[i, pl.ds(j * 16, 16)]
                    plsc.addupdate_scatter(cntloc, [d16], ones16)

        def drain_scatter(b):
            pltpu.make_async_copy(zdrain, rows[b], ss[b]).wait()

        def scatter(cc, gdesc):
            gdesc[cc].wait()
            gset = (cc // CPG) % 2
            pltpu.async_copy(
                rows[cc % NBUF], acc.at[dstv[gset].at[cc % CPG]],
                ss[cc % NBUF], add=True,
            )

        pltpu.sync_copy(e5.at[0, wid, 0], srcv[0])
        pltpu.sync_copy(e5.at[1, wid, 0], dstv[0])
        gdesc = [None] * CH
        pf = None
        for g in range(GRP):
            cur = g % 2
            if pf is not None:
                pf[0].wait()
                pf[1].wait()
            pf = None
            for k in range(CPG):
                cc = g * CPG + k
                b = cc % NBUF
                if cc >= NBUF:
                    drain_scatter(b)
                gdesc[cc] = pltpu.async_copy(
                    tbl.at[srcv[cur].at[k]], rows[b], sg[b]
                )
                count16(dstv[cur], k)
                if cc >= LAG:
                    scatter(cc - LAG, gdesc)
            if g + 1 < GRP:
                nxt = 1 - cur
                pf = (
                    pltpu.async_copy(e5.at[0, wid, g + 1], srcv[nxt], si[0]),
                    pltpu.async_copy(e5.at[1, wid, g + 1], dstv[nxt], si[1]),
                )
        for cc in range(CH - LAG, CH):
            scatter(cc, gdesc)
        for b in range(NBUF):
            drain_scatter(b)
        plsc.subcore_barrier()
        sl = pl.ds(s * rows_per_sub, rows_per_sub)
        pltpu.sync_copy(acc.at[sl], out.at[c, sl])
        if with_cnt:
            pltpu.sync_copy(cntloc, cnt_out.at[c, s])

    return agg


def _mid_body(p_ref, cntp_ref, w_ref, b_ref, g_ref, be_ref, y_ref, cnt_ref):
    n = y_ref.shape[0]
    sx = p_ref[0, :n, :] + p_ref[1, :n, :]
    dn = (((0,), (0,)), ((), ()))
    nw = cntp_ref.shape[0]
    cnt_full = lax.dot_general(
        cntp_ref[...],
        jnp.ones((nw, 1), jnp.float32),
        dn,
        preferred_element_type=jnp.float32,
    )
    cnt = cnt_full[:n, :]
    sy = jnp.dot(sx, w_ref[...], preferred_element_type=jnp.float32)
    h = (sy + cnt * b_ref[...]) / jnp.maximum(cnt, 1.0)
    m = jnp.mean(h, axis=0, keepdims=True)
    v = jnp.mean((h - m) ** 2, axis=0, keepdims=True)
    hn = (h - m) * lax.rsqrt(v + EPS) * g_ref[...] + be_ref[...]
    y_ref[...] = jnp.where(hn > 0, hn, jnp.exp(jnp.minimum(hn, 0.0)) - 1.0)
    cnt_ref[...] = cnt


def _final_body(p_ref, cnt_ref, w_ref, b_ref, g_ref, be_ref, batch_ref, wo_ref, bo_ref, o_ref):
    nn = batch_ref.shape[0]
    sx = p_ref[0, :nn, :] + p_ref[1, :nn, :]
    cnt = cnt_ref[...]
    sy = jnp.dot(sx, w_ref[...], preferred_element_type=jnp.float32)
    h = (sy + cnt * b_ref[...]) / jnp.maximum(cnt, 1.0)
    m = jnp.mean(h, axis=0, keepdims=True)
    v = jnp.mean((h - m) ** 2, axis=0, keepdims=True)
    hn = (h - m) * lax.rsqrt(v + EPS) * g_ref[...] + be_ref[...]
    h2 = jnp.where(hn > 0, hn, jnp.exp(jnp.minimum(hn, 0.0)) - 1.0)
    g = o_ref.shape[0]
    oh = (batch_ref[...] == lax.broadcasted_iota(jnp.int32, (nn, g), 1))
    oh = oh.astype(jnp.float32)
    dn = (((0,), (0,)), ((), ()))
    ps = lax.dot_general(oh, h2, dn, preferred_element_type=jnp.float32)
    pc = lax.dot_general(
        oh, jnp.ones((nn, 1), jnp.float32), dn, preferred_element_type=jnp.float32
    )
    pooled = ps / jnp.maximum(pc, 1.0)
    out = jnp.dot(pooled, wo_ref[...], preferred_element_type=jnp.float32)
    o_ref[...] = out + bo_ref[...]


def kernel(data, edge_index, batch, W1, b1, g1, be1, W2, b2, g2, be2, Wout, bout):
    n, d = data.shape
    h_dim = W1.shape[1]
    out_dim = Wout.shape[1]
    g_graphs = 64

    e5 = edge_index.reshape(2, NW, GRP, CPG, K)
    zdrain = jnp.zeros((K, h_dim), jnp.float32)

    p1, cntp = _make_agg(h_dim, True, 3)(data, e5, zdrain)

    x2, cnt = pl.pallas_call(
        _mid_body,
        out_shape=[
            jax.ShapeDtypeStruct((n, h_dim), jnp.float32),
            jax.ShapeDtypeStruct((n, 1), jnp.float32),
        ],
    )(
        p1,
        cntp.reshape(NW, NPAD),
        W1,
        b1.reshape(1, h_dim),
        g1.reshape(1, h_dim),
        be1.reshape(1, h_dim),
    )

    (p2,) = _make_agg(h_dim, False, 4)(x2, e5, zdrain)

    out = pl.pallas_call(
        _final_body,
        out_shape=jax.ShapeDtypeStruct((g_graphs, out_dim), jnp.float32),
    )(
        p2,
        cnt,
        W2,
        b2.reshape(1, h_dim),
        g2.reshape(1, h_dim),
        be2.reshape(1, h_dim),
        batch.reshape(n, 1),
        Wout,
        bout.reshape(1, out_dim),
    )
    return out


# speedup vs baseline: 1.1026x; 1.0002x over previous
import functools

import jax
import jax.numpy as jnp
from jax import lax
from jax.experimental import pallas as pl
from jax.experimental.pallas import tpu as pltpu
from jax.experimental.pallas import tpu_sc as plsc

EPS = 1e-5

NC = 2
NS = 16
NW = NC * NS

CH = 125
K = 80
GRP = 25
CPG = CH // GRP


NPAD = 10240


def _make_agg(width, with_cnt, NBUF):
    LAG = NBUF - 1
    rows_per_sub = NPAD // NS
    mesh = plsc.VectorSubcoreMesh(core_axis_name="c", subcore_axis_name="s")

    out_type = [jax.ShapeDtypeStruct((NC, NPAD, width), jnp.float32)]
    scratch = [pltpu.VMEM((CPG, K), jnp.int32) for _ in range(4)]
    scratch += [pltpu.VMEM((K, width), jnp.float32) for _ in range(NBUF)]
    scratch += [pltpu.VMEM_SHARED((NPAD, width), jnp.float32)]
    scratch += [pltpu.SemaphoreType.DMA for _ in range(2 * NBUF + 2)]
    if with_cnt:
        out_type.append(jax.ShapeDtypeStruct((NC, NS, NPAD), jnp.float32))
        scratch.append(pltpu.VMEM((NPAD,), jnp.float32))

    @functools.partial(
        pl.kernel,
        out_type=out_type,
        mesh=mesh,
        scratch_types=scratch,
        compiler_params=pltpu.CompilerParams(needs_layout_passes=False),
    )
    def agg(tbl, e5, zdrain, out, *rest):
        if with_cnt:
            cnt_out = rest[0]
            rest = rest[1:]
        srcv = rest[0:2]
        dstv = rest[2:4]
        rows = rest[4:4 + NBUF]
        acc = rest[4 + NBUF]
        sg = rest[5 + NBUF:5 + 2 * NBUF]
        ss = rest[5 + 2 * NBUF:5 + 3 * NBUF]
        si = rest[5 + 3 * NBUF:7 + 3 * NBUF]
        if with_cnt:
            cntloc = rest[7 + 3 * NBUF]
        c = lax.axis_index("c")
        s = lax.axis_index("s")
        wid = c * NS + s
        zv = jnp.zeros((16,), jnp.float32)

        def zrow(r, carry):
            for cc in range(width // 16):
                rows[0][r, pl.ds(cc * 16, 16)] = zv
            return carry

        lax.fori_loop(0, K, zrow, 0)
        for kk in range(rows_per_sub // K):
            pltpu.sync_copy(
                rows[0], acc.at[pl.ds(s * rows_per_sub + kk * K, K)]
            )
        if with_cnt:

            def zbody(i, carry):
                cntloc[pl.ds(i * 16, 16)] = zv
                return carry

            lax.fori_loop(0, NPAD // 16, zbody, 0)
        plsc.subcore_barrier()

        ones16 = jnp.ones((16,), jnp.float32)

        def count16(dv, i):
            if with_cnt:
                for j in range(K // 16):
                    d16 = dv[i, pl.ds(j * 16, 16)]
                    plsc.addupdate_scatter(cntloc, [d16], ones16)

        def drain_scatter(b):
            pltpu.make_async_copy(zdrain, rows[b], ss[b]).wait()

        def scatter(cc, gdesc):
            gdesc[cc].wait()
            gset = (cc // CPG) % 2
            pltpu.async_copy(
                rows[cc % NBUF], acc.at[dstv[gset].at[cc % CPG]],
                ss[cc % NBUF], add=True,
            )

        pltpu.sync_copy(e5.at[0, wid, 0], srcv[0])
        pltpu.sync_copy(e5.at[1, wid, 0], dstv[0])
        gdesc = [None] * CH
        pf = None
        for g in range(GRP):
            cur = g % 2
            if pf is not None:
                pf[0].wait()
                pf[1].wait()
            pf = None
            for k in range(CPG):
                cc = g * CPG + k
                b = cc % NBUF
                if cc >= NBUF:
                    drain_scatter(b)
                gdesc[cc] = pltpu.async_copy(
                    tbl.at[srcv[cur].at[k]], rows[b], sg[b]
                )
                count16(dstv[cur], k)
                if cc >= LAG:
                    scatter(cc - LAG, gdesc)
            if g + 1 < GRP:
                nxt = 1 - cur
                pf = (
                    pltpu.async_copy(e5.at[0, wid, g + 1], srcv[nxt], si[0]),
                    pltpu.async_copy(e5.at[1, wid, g + 1], dstv[nxt], si[1]),
                )
        for cc in range(CH - LAG, CH):
            scatter(cc, gdesc)
        for b in range(NBUF):
            drain_scatter(b)
        plsc.subcore_barrier()
        sl = pl.ds(s * rows_per_sub, rows_per_sub)
        pltpu.sync_copy(acc.at[sl], out.at[c, sl])
        if with_cnt:
            pltpu.sync_copy(cntloc, cnt_out.at[c, s])

    return agg


def _mid_body(p_ref, cntp_ref, w_ref, b_ref, g_ref, be_ref, y_ref, cnt_ref):
    n = y_ref.shape[0]
    sx = p_ref[0, :n, :] + p_ref[1, :n, :]
    dn = (((0,), (0,)), ((), ()))
    nw = cntp_ref.shape[0] * cntp_ref.shape[1]
    cnt_full = lax.dot_general(
        cntp_ref[...].reshape(nw, cntp_ref.shape[2]),
        jnp.ones((nw, 1), jnp.float32),
        dn,
        preferred_element_type=jnp.float32,
    )
    cnt = cnt_full[:n, :]
    sy = jnp.dot(sx, w_ref[...], preferred_element_type=jnp.float32)
    h = (sy + cnt * b_ref[...]) / jnp.maximum(cnt, 1.0)
    m = jnp.mean(h, axis=0, keepdims=True)
    v = jnp.mean((h - m) ** 2, axis=0, keepdims=True)
    hn = (h - m) * lax.rsqrt(v + EPS) * g_ref[...] + be_ref[...]
    y_ref[...] = jnp.where(hn > 0, hn, jnp.exp(jnp.minimum(hn, 0.0)) - 1.0)
    cnt_ref[...] = cnt


def _final_body(p_ref, cnt_ref, w_ref, b_ref, g_ref, be_ref, batch_ref, wo_ref, bo_ref, o_ref):
    nn = batch_ref.shape[0]
    sx = p_ref[0, :nn, :] + p_ref[1, :nn, :]
    cnt = cnt_ref[...]
    sy = jnp.dot(sx, w_ref[...], preferred_element_type=jnp.float32)
    h = (sy + cnt * b_ref[...]) / jnp.maximum(cnt, 1.0)
    m = jnp.mean(h, axis=0, keepdims=True)
    v = jnp.mean((h - m) ** 2, axis=0, keepdims=True)
    hn = (h - m) * lax.rsqrt(v + EPS) * g_ref[...] + be_ref[...]
    h2 = jnp.where(hn > 0, hn, jnp.exp(jnp.minimum(hn, 0.0)) - 1.0)
    g = o_ref.shape[0]
    oh = (batch_ref[...] == lax.broadcasted_iota(jnp.int32, (nn, g), 1))
    oh = oh.astype(jnp.float32)
    dn = (((0,), (0,)), ((), ()))
    ps = lax.dot_general(oh, h2, dn, preferred_element_type=jnp.float32)
    pc = lax.dot_general(
        oh, jnp.ones((nn, 1), jnp.float32), dn, preferred_element_type=jnp.float32
    )
    pooled = ps / jnp.maximum(pc, 1.0)
    out = jnp.dot(pooled, wo_ref[...], preferred_element_type=jnp.float32)
    o_ref[...] = out + bo_ref[...]


def kernel(data, edge_index, batch, W1, b1, g1, be1, W2, b2, g2, be2, Wout, bout):
    n, d = data.shape
    h_dim = W1.shape[1]
    out_dim = Wout.shape[1]
    g_graphs = 64

    e5 = edge_index.reshape(2, NW, GRP, CPG, K)
    zdrain = jnp.zeros((K, h_dim), jnp.float32)

    p1, cntp = _make_agg(h_dim, True, 3)(data, e5, zdrain)

    x2, cnt = pl.pallas_call(
        _mid_body,
        out_shape=[
            jax.ShapeDtypeStruct((n, h_dim), jnp.float32),
            jax.ShapeDtypeStruct((n, 1), jnp.float32),
        ],
    )(
        p1,
        cntp,
        W1,
        b1.reshape(1, h_dim),
        g1.reshape(1, h_dim),
        be1.reshape(1, h_dim),
    )

    (p2,) = _make_agg(h_dim, False, 4)(x2, e5, zdrain)

    out = pl.pallas_call(
        _final_body,
        out_shape=jax.ShapeDtypeStruct((g_graphs, out_dim), jnp.float32),
    )(
        p2,
        cnt,
        W2,
        b2.reshape(1, h_dim),
        g2.reshape(1, h_dim),
        be2.reshape(1, h_dim),
        batch.reshape(n, 1),
        Wout,
        bout.reshape(1, out_dim),
    )
    return out
